# baseline (speedup 1.0000x reference)
import jax
import jax.numpy as jnp
from jax.experimental import pallas as pl
from jax.experimental.pallas import tpu as pltpu

_N = 16777216
_ROWS = 8192
_COLS = 2048
_BLOCK_ROWS = 1024


def _binarize_body(x_ref, o_ref):
    x = x_ref[...]
    y = jnp.where(x <= 1.0, 0.0, x)
    o_ref[...] = jnp.where(y > 1.0, 1.0, y)


def kernel(x):
    x2 = x.reshape(_ROWS, _COLS)
    out = pl.pallas_call(
        _binarize_body,
        grid=(_ROWS // _BLOCK_ROWS,),
        in_specs=[pl.BlockSpec((_BLOCK_ROWS, _COLS), lambda i: (i, 0))],
        out_specs=pl.BlockSpec((_BLOCK_ROWS, _COLS), lambda i: (i, 0)),
        out_shape=jax.ShapeDtypeStruct((_ROWS, _COLS), jnp.float32),
        compiler_params=pltpu.CompilerParams(
            dimension_semantics=("arbitrary",),
        ),
    )(x2)
    return out.reshape(_N)


# speedup vs baseline: 4.0613x; 4.0613x over previous
import jax
import jax.numpy as jnp
from jax.experimental import pallas as pl
from jax.experimental.pallas import tpu as pltpu

_N = 16777216
_ROWS = 8192
_COLS = 2048
_BLOCK_ROWS = 1024


def _binarize_body(x_ref, o_ref):
    x = x_ref[...]
    y = jnp.where(x <= 1.0, 0.0, x)
    o_ref[...] = jnp.where(y > 1.0, 1.0, y)


_BLOCK = 2097152


def kernel(x):
    return pl.pallas_call(
        _binarize_body,
        grid=(_N // _BLOCK,),
        in_specs=[pl.BlockSpec((_BLOCK,), lambda i: (i,))],
        out_specs=pl.BlockSpec((_BLOCK,), lambda i: (i,)),
        out_shape=jax.ShapeDtypeStruct((_N,), jnp.float32),
        compiler_params=pltpu.CompilerParams(
            dimension_semantics=("arbitrary",),
        ),
    )(x)
